# baseline (speedup 1.0000x reference)
import functools

import jax
import jax.numpy as jnp
from jax import lax
from jax.experimental import pallas as pl
from jax.experimental.pallas import tpu as pltpu
from jax.experimental.pallas import tpu_sc as plsc

EMBED = 768
FFN = 3072
E = 8
SEQ = 2048
TOPK = 2

R = 256
NBLK = (TOPK * SEQ + E * (R - 1) + R - 1) // R
P = NBLK * R
NC = 2
NS = 16
NW = NC * NS
TPW = SEQ // NW
WLANE = 16

_INV_SQRT2 = 0.7071067811865476


def _route_body(x_ref, wr_ref, tri_ref, pos1_ref, pos2_ref, w1b_ref, w2b_ref,
                bexp_ref, nact_ref):
    x = x_ref[...]
    wr = wr_ref[...]
    logits = jnp.dot(x, wr, preferred_element_type=jnp.float32)
    m = jnp.max(logits, axis=-1, keepdims=True)
    ex = jnp.exp(logits - m)
    probs = ex / jnp.sum(ex, axis=-1, keepdims=True)

    idx = lax.broadcasted_iota(jnp.int32, (SEQ, E), 1)
    p1 = jnp.max(probs, axis=-1, keepdims=True)
    i1 = jnp.min(jnp.where(probs >= p1, idx, E), axis=-1, keepdims=True)
    oh1 = idx == i1
    probs2 = jnp.where(oh1, -1.0, probs)
    p2 = jnp.max(probs2, axis=-1, keepdims=True)
    i2 = jnp.min(jnp.where(probs2 >= p2, idx, E), axis=-1, keepdims=True)
    oh2 = idx == i2

    denom = p1 + p2 + 1e-8
    w1b_ref[...] = jnp.broadcast_to(p1 / denom, (SEQ, WLANE))
    w2b_ref[...] = jnp.broadcast_to(p2 / denom, (SEQ, WLANE))

    f1 = oh1.astype(jnp.bfloat16)
    f2 = oh2.astype(jnp.bfloat16)
    tri = tri_ref[...]
    c1 = jnp.dot(tri, f1, preferred_element_type=jnp.float32)
    c2 = jnp.dot(tri, f2, preferred_element_type=jnp.float32)
    cnt1 = c1[SEQ - 1:SEQ, :]
    cnt2 = c2[SEQ - 1:SEQ, :]
    cnt = cnt1 + cnt2

    nblk_e = jnp.floor((cnt + (R - 1)) * (1.0 / R))
    up = (lax.broadcasted_iota(jnp.int32, (E, E), 0)
          < lax.broadcasted_iota(jnp.int32, (E, E), 1)).astype(jnp.float32)
    blk_off = jnp.dot(nblk_e, up, preferred_element_type=jnp.float32,
                      precision=lax.Precision.HIGHEST)
    off = blk_off * R

    pos1 = jnp.sum(jnp.where(oh1, off + c1 - 1.0, 0.0), axis=1, keepdims=True)
    pos2 = jnp.sum(jnp.where(oh2, off + cnt1 + c2 - 1.0, 0.0), axis=1,
                   keepdims=True)
    pos1_ref[...] = pos1.astype(jnp.int32)
    pos2_ref[...] = pos2.astype(jnp.int32)

    blk_end = blk_off + nblk_e
    bb = lax.broadcasted_iota(jnp.int32, (NBLK, E), 0).astype(jnp.float32)
    be = jnp.sum((bb >= blk_end).astype(jnp.float32), axis=1, keepdims=True)
    bexp_ref[...] = jnp.minimum(be, E - 1.0).astype(jnp.int32)
    nact_ref[...] = jnp.sum(nblk_e, axis=1, keepdims=True).astype(jnp.int32)


def _dispatch_body(x_hbm, pos1_hbm, pos2_hbm, xs_hbm,
                   rows_v, idx1_v, idx2_v, sem):
    wid = lax.axis_index("s") * NC + lax.axis_index("c")
    base = wid * TPW
    pltpu.sync_copy(x_hbm.at[pl.ds(base, TPW)], rows_v)
    pltpu.sync_copy(pos1_hbm.at[pl.ds(base, TPW)], idx1_v)
    pltpu.sync_copy(pos2_hbm.at[pl.ds(base, TPW)], idx2_v)
    cp1 = pltpu.async_copy(rows_v, xs_hbm.at[idx1_v], sem)
    cp2 = pltpu.async_copy(rows_v, xs_hbm.at[idx2_v], sem)
    cp1.wait()
    cp2.wait()


@functools.cache
def _make_dispatch():
    mesh = plsc.VectorSubcoreMesh(core_axis_name="c", subcore_axis_name="s",
                                  num_cores=NC, num_subcores=NS)
    return pl.kernel(
        _dispatch_body,
        out_type=jax.ShapeDtypeStruct((P, EMBED), jnp.float32),
        mesh=mesh,
        scratch_types=[pltpu.VMEM((TPW, EMBED), jnp.float32),
                       pltpu.VMEM((TPW,), jnp.int32),
                       pltpu.VMEM((TPW,), jnp.int32),
                       pltpu.SemaphoreType.DMA],
    )


def _gemm_body(bexp_ref, nact_ref, xs_ref, w1_ref, b1_ref, w2_ref, b2_ref,
               y_ref):
    b = pl.program_id(0)

    @pl.when(b < nact_ref[0, 0])
    def _():
        xb = xs_ref[...].astype(jnp.bfloat16)
        h = jnp.dot(xb, w1_ref[0].astype(jnp.bfloat16),
                    preferred_element_type=jnp.float32) + b1_ref[0]
        h = 0.5 * h * (1.0 + lax.erf(h * _INV_SQRT2))
        y = jnp.dot(h.astype(jnp.bfloat16), w2_ref[0].astype(jnp.bfloat16),
                    preferred_element_type=jnp.float32) + b2_ref[0]
        y_ref[...] = y


_TRI = None


def _tri_const():
    global _TRI
    if _TRI is None:
        import numpy as np
        _TRI = jnp.asarray(np.tril(np.ones((SEQ, SEQ), np.float32))
                           .astype(jnp.bfloat16))
    return _TRI


def _active(b, nact):
    return jnp.minimum(b, nact[0, 0] - 1)


_GEMM_GRID = pltpu.PrefetchScalarGridSpec(
    num_scalar_prefetch=2,
    grid=(NBLK,),
    in_specs=[
        pl.BlockSpec((R, EMBED), lambda b, bexp, nact: (_active(b, nact), 0)),
        pl.BlockSpec((1, EMBED, FFN),
                     lambda b, bexp, nact: (bexp[_active(b, nact), 0], 0, 0)),
        pl.BlockSpec((1, 1, FFN),
                     lambda b, bexp, nact: (bexp[_active(b, nact), 0], 0, 0)),
        pl.BlockSpec((1, FFN, EMBED),
                     lambda b, bexp, nact: (bexp[_active(b, nact), 0], 0, 0)),
        pl.BlockSpec((1, 1, EMBED),
                     lambda b, bexp, nact: (bexp[_active(b, nact), 0], 0, 0)),
    ],
    out_specs=pl.BlockSpec((R, EMBED), lambda b, bexp, nact: (_active(b, nact), 0)),
)


def _combine_body(y_hbm, pos1_hbm, pos2_hbm, w1b_hbm, w2b_hbm, out_hbm,
                  buf1, buf2, w1_v, w2_v, idx1_v, idx2_v, sem):
    wid = lax.axis_index("s") * NC + lax.axis_index("c")
    base = wid * TPW
    pltpu.sync_copy(pos1_hbm.at[pl.ds(base, TPW)], idx1_v)
    pltpu.sync_copy(pos2_hbm.at[pl.ds(base, TPW)], idx2_v)
    pltpu.sync_copy(w1b_hbm.at[pl.ds(base, TPW)], w1_v)
    pltpu.sync_copy(w2b_hbm.at[pl.ds(base, TPW)], w2_v)
    g1 = pltpu.async_copy(y_hbm.at[idx1_v], buf1, sem)
    g2 = pltpu.async_copy(y_hbm.at[idx2_v], buf2, sem)
    g1.wait()
    g2.wait()

    def row(t, carry):
        wv1 = w1_v[t, :]
        wv2 = w2_v[t, :]
        for j in range(EMBED // 16):
            sl = pl.ds(j * 16, 16)
            buf1[t, sl] = buf1[t, sl] * wv1 + buf2[t, sl] * wv2
        return carry

    lax.fori_loop(0, TPW, row, 0)
    pltpu.sync_copy(buf1, out_hbm.at[pl.ds(base, TPW)])


@functools.cache
def _make_combine():
    mesh = plsc.VectorSubcoreMesh(core_axis_name="c", subcore_axis_name="s",
                                  num_cores=NC, num_subcores=NS)
    return pl.kernel(
        _combine_body,
        out_type=jax.ShapeDtypeStruct((SEQ, EMBED), jnp.float32),
        mesh=mesh,
        scratch_types=[pltpu.VMEM((TPW, EMBED), jnp.float32),
                       pltpu.VMEM((TPW, EMBED), jnp.float32),
                       pltpu.VMEM((TPW, WLANE), jnp.float32),
                       pltpu.VMEM((TPW, WLANE), jnp.float32),
                       pltpu.VMEM((TPW,), jnp.int32),
                       pltpu.VMEM((TPW,), jnp.int32),
                       pltpu.SemaphoreType.DMA],
    )


def kernel(x, Wr, W1, b1, W2, b2):
    xs0 = x.reshape(SEQ, EMBED)
    pos1_2, pos2_2, w1b, w2b, bexp_2, nact_2 = pl.pallas_call(
        _route_body,
        out_shape=[jax.ShapeDtypeStruct((SEQ, 1), jnp.int32),
                   jax.ShapeDtypeStruct((SEQ, 1), jnp.int32),
                   jax.ShapeDtypeStruct((SEQ, WLANE), jnp.float32),
                   jax.ShapeDtypeStruct((SEQ, WLANE), jnp.float32),
                   jax.ShapeDtypeStruct((NBLK, 1), jnp.int32),
                   jax.ShapeDtypeStruct((1, 1), jnp.int32)],
    )(xs0, Wr, _tri_const())
    pos1 = pos1_2.reshape(SEQ)
    pos2 = pos2_2.reshape(SEQ)

    xs = _make_dispatch()(xs0, pos1, pos2)

    y = pl.pallas_call(
        _gemm_body,
        grid_spec=_GEMM_GRID,
        out_shape=jax.ShapeDtypeStruct((P, EMBED), jnp.float32),
    )(bexp_2, nact_2, xs, W1, b1.reshape(E, 1, FFN), W2, b2.reshape(E, 1, EMBED))

    out = _make_combine()(y, pos1, pos2, w1b, w2b)
    return out.reshape(x.shape)


# speedup vs baseline: 1.0905x; 1.0905x over previous
import functools

import jax
import jax.numpy as jnp
from jax import lax
from jax.experimental import pallas as pl
from jax.experimental.pallas import tpu as pltpu
from jax.experimental.pallas import tpu_sc as plsc

EMBED = 768
FFN = 3072
E = 8
SEQ = 2048
TOPK = 2

R = 512
NBLK = (TOPK * SEQ + E * (R - 1) + R - 1) // R
P = NBLK * R
NC = 2
NS = 16
NW = NC * NS
TPW = SEQ // NW
WLANE = 16

_INV_SQRT2 = 0.7071067811865476


def _route_body(x_ref, wr_ref, tri_ref, pos1_ref, pos2_ref, w1b_ref, w2b_ref,
                bexp_ref, nact_ref):
    x = x_ref[...]
    wr = wr_ref[...]
    logits = jnp.dot(x, wr, preferred_element_type=jnp.float32)
    m = jnp.max(logits, axis=-1, keepdims=True)
    ex = jnp.exp(logits - m)
    probs = ex / jnp.sum(ex, axis=-1, keepdims=True)

    idx = lax.broadcasted_iota(jnp.int32, (SEQ, E), 1)
    p1 = jnp.max(probs, axis=-1, keepdims=True)
    i1 = jnp.min(jnp.where(probs >= p1, idx, E), axis=-1, keepdims=True)
    oh1 = idx == i1
    probs2 = jnp.where(oh1, -1.0, probs)
    p2 = jnp.max(probs2, axis=-1, keepdims=True)
    i2 = jnp.min(jnp.where(probs2 >= p2, idx, E), axis=-1, keepdims=True)
    oh2 = idx == i2

    denom = p1 + p2 + 1e-8
    w1b_ref[...] = jnp.broadcast_to(p1 / denom, (SEQ, WLANE))
    w2b_ref[...] = jnp.broadcast_to(p2 / denom, (SEQ, WLANE))

    f1 = oh1.astype(jnp.bfloat16)
    f2 = oh2.astype(jnp.bfloat16)
    tri = tri_ref[...]
    c1 = jnp.dot(tri, f1, preferred_element_type=jnp.float32)
    c2 = jnp.dot(tri, f2, preferred_element_type=jnp.float32)
    cnt1 = c1[SEQ - 1:SEQ, :]
    cnt2 = c2[SEQ - 1:SEQ, :]
    cnt = cnt1 + cnt2

    nblk_e = jnp.floor((cnt + (R - 1)) * (1.0 / R))
    up = (lax.broadcasted_iota(jnp.int32, (E, E), 0)
          < lax.broadcasted_iota(jnp.int32, (E, E), 1)).astype(jnp.float32)
    blk_off = jnp.dot(nblk_e, up, preferred_element_type=jnp.float32,
                      precision=lax.Precision.HIGHEST)
    off = blk_off * R

    pos1 = jnp.sum(jnp.where(oh1, off + c1 - 1.0, 0.0), axis=1, keepdims=True)
    pos2 = jnp.sum(jnp.where(oh2, off + cnt1 + c2 - 1.0, 0.0), axis=1,
                   keepdims=True)
    pos1_ref[...] = pos1.astype(jnp.int32)
    pos2_ref[...] = pos2.astype(jnp.int32)

    blk_end = blk_off + nblk_e
    bb = lax.broadcasted_iota(jnp.int32, (NBLK, E), 0).astype(jnp.float32)
    be = jnp.sum((bb >= blk_end).astype(jnp.float32), axis=1, keepdims=True)
    bexp_ref[...] = jnp.minimum(be, E - 1.0).astype(jnp.int32)
    nact_ref[...] = jnp.sum(nblk_e, axis=1, keepdims=True).astype(jnp.int32)


def _dispatch_body(x_hbm, pos1_hbm, pos2_hbm, xs_hbm,
                   rows_v, idx1_v, idx2_v, sem):
    wid = lax.axis_index("s") * NC + lax.axis_index("c")
    base = wid * TPW
    rcp = pltpu.async_copy(x_hbm.at[pl.ds(base, TPW)], rows_v, sem)
    pltpu.sync_copy(pos1_hbm.at[pl.ds(base, TPW)], idx1_v)
    pltpu.sync_copy(pos2_hbm.at[pl.ds(base, TPW)], idx2_v)
    rcp.wait()
    cp1 = pltpu.async_copy(rows_v, xs_hbm.at[idx1_v], sem)
    cp2 = pltpu.async_copy(rows_v, xs_hbm.at[idx2_v], sem)
    cp1.wait()
    cp2.wait()


@functools.cache
def _make_dispatch():
    mesh = plsc.VectorSubcoreMesh(core_axis_name="c", subcore_axis_name="s",
                                  num_cores=NC, num_subcores=NS)
    return pl.kernel(
        _dispatch_body,
        out_type=jax.ShapeDtypeStruct((P, EMBED), jnp.float32),
        mesh=mesh,
        scratch_types=[pltpu.VMEM((TPW, EMBED), jnp.float32),
                       pltpu.VMEM((TPW,), jnp.int32),
                       pltpu.VMEM((TPW,), jnp.int32),
                       pltpu.SemaphoreType.DMA],
    )


def _gemm_body(bexp_ref, nact_ref, xs_ref, w1_ref, b1_ref, w2_ref, b2_ref,
               y_ref):
    b = pl.program_id(0)

    @pl.when(b < nact_ref[0, 0])
    def _():
        xb = xs_ref[...].astype(jnp.bfloat16)
        h = jnp.dot(xb, w1_ref[0].astype(jnp.bfloat16),
                    preferred_element_type=jnp.float32) + b1_ref[0]
        h = 0.5 * h * (1.0 + lax.erf(h * _INV_SQRT2))
        y = jnp.dot(h.astype(jnp.bfloat16), w2_ref[0].astype(jnp.bfloat16),
                    preferred_element_type=jnp.float32) + b2_ref[0]
        y_ref[...] = y


_TRI = None


def _tri_const():
    global _TRI
    if _TRI is None:
        import numpy as np
        _TRI = jnp.asarray(np.tril(np.ones((SEQ, SEQ), np.float32))
                           .astype(jnp.bfloat16))
    return _TRI


def _active(b, nact):
    return jnp.minimum(b, nact[0, 0] - 1)


_GEMM_GRID = pltpu.PrefetchScalarGridSpec(
    num_scalar_prefetch=2,
    grid=(NBLK,),
    in_specs=[
        pl.BlockSpec((R, EMBED), lambda b, bexp, nact: (_active(b, nact), 0)),
        pl.BlockSpec((1, EMBED, FFN),
                     lambda b, bexp, nact: (bexp[_active(b, nact), 0], 0, 0)),
        pl.BlockSpec((1, 1, FFN),
                     lambda b, bexp, nact: (bexp[_active(b, nact), 0], 0, 0)),
        pl.BlockSpec((1, FFN, EMBED),
                     lambda b, bexp, nact: (bexp[_active(b, nact), 0], 0, 0)),
        pl.BlockSpec((1, 1, EMBED),
                     lambda b, bexp, nact: (bexp[_active(b, nact), 0], 0, 0)),
    ],
    out_specs=pl.BlockSpec((R, EMBED), lambda b, bexp, nact: (_active(b, nact), 0)),
)


def _combine_body(y_hbm, pos1_hbm, pos2_hbm, w1b_hbm, w2b_hbm, out_hbm,
                  buf1, buf2, w1_v, w2_v, idx1_v, idx2_v, sem, sem2):
    wid = lax.axis_index("s") * NC + lax.axis_index("c")
    base = wid * TPW
    pltpu.sync_copy(pos1_hbm.at[pl.ds(base, TPW)], idx1_v)
    pltpu.sync_copy(pos2_hbm.at[pl.ds(base, TPW)], idx2_v)
    pltpu.sync_copy(w1b_hbm.at[pl.ds(base, TPW)], w1_v)
    pltpu.sync_copy(w2b_hbm.at[pl.ds(base, TPW)], w2_v)
    g1 = pltpu.async_copy(y_hbm.at[idx1_v], buf1, sem)
    g2 = pltpu.async_copy(y_hbm.at[idx2_v], buf2, sem2)
    g1.wait()

    def row_scale(t, carry):
        wv1 = w1_v[t, :]
        for j in range(EMBED // 16):
            sl = pl.ds(j * 16, 16)
            buf1[t, sl] = buf1[t, sl] * wv1
        return carry

    lax.fori_loop(0, TPW, row_scale, 0)
    g2.wait()

    def row_acc(t, carry):
        wv2 = w2_v[t, :]
        for j in range(EMBED // 16):
            sl = pl.ds(j * 16, 16)
            buf1[t, sl] = buf1[t, sl] + buf2[t, sl] * wv2
        return carry

    lax.fori_loop(0, TPW, row_acc, 0)
    pltpu.sync_copy(buf1, out_hbm.at[pl.ds(base, TPW)])


@functools.cache
def _make_combine():
    mesh = plsc.VectorSubcoreMesh(core_axis_name="c", subcore_axis_name="s",
                                  num_cores=NC, num_subcores=NS)
    return pl.kernel(
        _combine_body,
        out_type=jax.ShapeDtypeStruct((SEQ, EMBED), jnp.float32),
        mesh=mesh,
        scratch_types=[pltpu.VMEM((TPW, EMBED), jnp.float32),
                       pltpu.VMEM((TPW, EMBED), jnp.float32),
                       pltpu.VMEM((TPW, WLANE), jnp.float32),
                       pltpu.VMEM((TPW, WLANE), jnp.float32),
                       pltpu.VMEM((TPW,), jnp.int32),
                       pltpu.VMEM((TPW,), jnp.int32),
                       pltpu.SemaphoreType.DMA,
                       pltpu.SemaphoreType.DMA],
    )


def kernel(x, Wr, W1, b1, W2, b2):
    xs0 = x.reshape(SEQ, EMBED)
    pos1_2, pos2_2, w1b, w2b, bexp_2, nact_2 = pl.pallas_call(
        _route_body,
        out_shape=[jax.ShapeDtypeStruct((SEQ, 1), jnp.int32),
                   jax.ShapeDtypeStruct((SEQ, 1), jnp.int32),
                   jax.ShapeDtypeStruct((SEQ, WLANE), jnp.float32),
                   jax.ShapeDtypeStruct((SEQ, WLANE), jnp.float32),
                   jax.ShapeDtypeStruct((NBLK, 1), jnp.int32),
                   jax.ShapeDtypeStruct((1, 1), jnp.int32)],
    )(xs0, Wr, _tri_const())
    pos1 = pos1_2.reshape(SEQ)
    pos2 = pos2_2.reshape(SEQ)

    xs = _make_dispatch()(xs0, pos1, pos2)

    y = pl.pallas_call(
        _gemm_body,
        grid_spec=_GEMM_GRID,
        out_shape=jax.ShapeDtypeStruct((P, EMBED), jnp.float32),
    )(bexp_2, nact_2, xs, W1, b1.reshape(E, 1, FFN), W2, b2.reshape(E, 1, EMBED))

    out = _make_combine()(y, pos1, pos2, w1b, w2b)
    return out.reshape(x.shape)


# speedup vs baseline: 1.1029x; 1.0113x over previous
import functools

import jax
import jax.numpy as jnp
from jax import lax
from jax.experimental import pallas as pl
from jax.experimental.pallas import tpu as pltpu
from jax.experimental.pallas import tpu_sc as plsc

EMBED = 768
FFN = 3072
E = 8
SEQ = 2048
TOPK = 2

R = 512
NBLK = (TOPK * SEQ + E * (R - 1) + R - 1) // R
P = NBLK * R
NC = 2
NS = 16
NW = NC * NS
TPW = SEQ // NW
WLANE = 16

_INV_SQRT2 = 0.7071067811865476


def _route_body(x_ref, wr_ref, tri_ref, pos1_ref, pos2_ref, w1b_ref, w2b_ref,
                bexp_ref, nact_ref):
    x = x_ref[...]
    wr = wr_ref[...]
    logits = jnp.dot(x, wr, preferred_element_type=jnp.float32)
    m = jnp.max(logits, axis=-1, keepdims=True)
    ex = jnp.exp(logits - m)
    probs = ex / jnp.sum(ex, axis=-1, keepdims=True)

    idx = lax.broadcasted_iota(jnp.int32, (SEQ, E), 1)
    p1 = jnp.max(probs, axis=-1, keepdims=True)
    i1 = jnp.min(jnp.where(probs >= p1, idx, E), axis=-1, keepdims=True)
    oh1 = idx == i1
    probs2 = jnp.where(oh1, -1.0, probs)
    p2 = jnp.max(probs2, axis=-1, keepdims=True)
    i2 = jnp.min(jnp.where(probs2 >= p2, idx, E), axis=-1, keepdims=True)
    oh2 = idx == i2

    denom = p1 + p2 + 1e-8
    w1b_ref[...] = jnp.broadcast_to(p1 / denom, (SEQ, WLANE))
    w2b_ref[...] = jnp.broadcast_to(p2 / denom, (SEQ, WLANE))

    f1 = oh1.astype(jnp.bfloat16)
    f2 = oh2.astype(jnp.bfloat16)
    tri = tri_ref[...]
    c1 = jnp.dot(tri, f1, preferred_element_type=jnp.float32)
    c2 = jnp.dot(tri, f2, preferred_element_type=jnp.float32)
    cnt1 = c1[SEQ - 1:SEQ, :]
    cnt2 = c2[SEQ - 1:SEQ, :]
    cnt = cnt1 + cnt2

    nblk_e = jnp.floor((cnt + (R - 1)) * (1.0 / R))
    up = (lax.broadcasted_iota(jnp.int32, (E, E), 0)
          < lax.broadcasted_iota(jnp.int32, (E, E), 1)).astype(jnp.float32)
    blk_off = jnp.dot(nblk_e, up, preferred_element_type=jnp.float32,
                      precision=lax.Precision.HIGHEST)
    off = blk_off * R

    pos1 = jnp.sum(jnp.where(oh1, off + c1 - 1.0, 0.0), axis=1, keepdims=True)
    pos2 = jnp.sum(jnp.where(oh2, off + cnt1 + c2 - 1.0, 0.0), axis=1,
                   keepdims=True)
    pos1_ref[...] = pos1.astype(jnp.int32)
    pos2_ref[...] = pos2.astype(jnp.int32)

    blk_end = blk_off + nblk_e
    bb = lax.broadcasted_iota(jnp.int32, (NBLK, E), 0).astype(jnp.float32)
    be = jnp.sum((bb >= blk_end).astype(jnp.float32), axis=1, keepdims=True)
    bexp_ref[...] = jnp.minimum(be, E - 1.0).astype(jnp.int32)
    nact_ref[...] = jnp.sum(nblk_e, axis=1, keepdims=True).astype(jnp.int32)


def _dispatch_body(x_hbm, pos1_hbm, pos2_hbm, xs_hbm,
                   rows_v, idx1_v, idx2_v, sem):
    wid = lax.axis_index("s") * NC + lax.axis_index("c")
    base = wid * TPW
    rcp = pltpu.async_copy(x_hbm.at[pl.ds(base, TPW)], rows_v, sem)
    pltpu.sync_copy(pos1_hbm.at[pl.ds(base, TPW)], idx1_v)
    pltpu.sync_copy(pos2_hbm.at[pl.ds(base, TPW)], idx2_v)
    rcp.wait()
    cp1 = pltpu.async_copy(rows_v, xs_hbm.at[idx1_v], sem)
    cp2 = pltpu.async_copy(rows_v, xs_hbm.at[idx2_v], sem)
    cp1.wait()
    cp2.wait()


@functools.cache
def _make_dispatch():
    mesh = plsc.VectorSubcoreMesh(core_axis_name="c", subcore_axis_name="s",
                                  num_cores=NC, num_subcores=NS)
    return pl.kernel(
        _dispatch_body,
        out_type=jax.ShapeDtypeStruct((P, EMBED), jnp.float32),
        mesh=mesh,
        scratch_types=[pltpu.VMEM((TPW, EMBED), jnp.float32),
                       pltpu.VMEM((TPW,), jnp.int32),
                       pltpu.VMEM((TPW,), jnp.int32),
                       pltpu.SemaphoreType.DMA],
    )


def _gemm_body(bexp_ref, nact_ref, xs_ref, w1_ref, b1_ref, w2_ref, b2_ref,
               y_ref):
    b = pl.program_id(0)

    @pl.when(b < nact_ref[0, 0])
    def _():
        xb = xs_ref[...].astype(jnp.bfloat16)
        h = jnp.dot(xb, w1_ref[0].astype(jnp.bfloat16),
                    preferred_element_type=jnp.float32) + b1_ref[0]
        h = 0.5 * h * (1.0 + lax.erf(h * _INV_SQRT2))
        y = jnp.dot(h.astype(jnp.bfloat16), w2_ref[0].astype(jnp.bfloat16),
                    preferred_element_type=jnp.float32) + b2_ref[0]
        y_ref[...] = y


_TRI = None


def _tri_const():
    global _TRI
    if _TRI is None:
        import numpy as np
        _TRI = jnp.asarray(np.tril(np.ones((SEQ, SEQ), np.float32))
                           .astype(jnp.bfloat16))
    return _TRI


def _active(b, nact):
    return jnp.minimum(b, nact[0, 0] - 1)


_GEMM_GRID = pltpu.PrefetchScalarGridSpec(
    num_scalar_prefetch=2,
    grid=(NBLK,),
    in_specs=[
        pl.BlockSpec((R, EMBED), lambda b, bexp, nact: (_active(b, nact), 0)),
        pl.BlockSpec((1, EMBED, FFN),
                     lambda b, bexp, nact: (bexp[_active(b, nact), 0], 0, 0)),
        pl.BlockSpec((1, 1, FFN),
                     lambda b, bexp, nact: (bexp[_active(b, nact), 0], 0, 0)),
        pl.BlockSpec((1, FFN, EMBED),
                     lambda b, bexp, nact: (bexp[_active(b, nact), 0], 0, 0)),
        pl.BlockSpec((1, 1, EMBED),
                     lambda b, bexp, nact: (bexp[_active(b, nact), 0], 0, 0)),
    ],
    out_specs=pl.BlockSpec((R, EMBED), lambda b, bexp, nact: (_active(b, nact), 0)),
)


def _combine_body(y_hbm, pos1_hbm, pos2_hbm, w1b_hbm, w2b_hbm, out_hbm,
                  buf1, buf2, w1_v, w2_v, idx1_v, idx2_v, sem, sem2):
    wid = lax.axis_index("s") * NC + lax.axis_index("c")
    base = wid * TPW
    pltpu.sync_copy(pos1_hbm.at[pl.ds(base, TPW)], idx1_v)
    pltpu.sync_copy(pos2_hbm.at[pl.ds(base, TPW)], idx2_v)
    pltpu.sync_copy(w1b_hbm.at[pl.ds(base, TPW)], w1_v)
    pltpu.sync_copy(w2b_hbm.at[pl.ds(base, TPW)], w2_v)
    g1 = pltpu.async_copy(y_hbm.at[idx1_v], buf1, sem)
    g2 = pltpu.async_copy(y_hbm.at[idx2_v], buf2, sem2)
    g1.wait()
    g2.wait()

    def row(t, carry):
        wv1 = w1_v[t, :]
        wv2 = w2_v[t, :]
        for j in range(EMBED // 16):
            sl = pl.ds(j * 16, 16)
            buf1[t, sl] = buf1[t, sl] * wv1 + buf2[t, sl] * wv2
        return carry

    lax.fori_loop(0, TPW, row, 0)
    pltpu.sync_copy(buf1, out_hbm.at[pl.ds(base, TPW)])


@functools.cache
def _make_combine():
    mesh = plsc.VectorSubcoreMesh(core_axis_name="c", subcore_axis_name="s",
                                  num_cores=NC, num_subcores=NS)
    return pl.kernel(
        _combine_body,
        out_type=jax.ShapeDtypeStruct((SEQ, EMBED), jnp.float32),
        mesh=mesh,
        scratch_types=[pltpu.VMEM((TPW, EMBED), jnp.float32),
                       pltpu.VMEM((TPW, EMBED), jnp.float32),
                       pltpu.VMEM((TPW, WLANE), jnp.float32),
                       pltpu.VMEM((TPW, WLANE), jnp.float32),
                       pltpu.VMEM((TPW,), jnp.int32),
                       pltpu.VMEM((TPW,), jnp.int32),
                       pltpu.SemaphoreType.DMA,
                       pltpu.SemaphoreType.DMA],
    )


def kernel(x, Wr, W1, b1, W2, b2):
    xs0 = x.reshape(SEQ, EMBED)
    pos1_2, pos2_2, w1b, w2b, bexp_2, nact_2 = pl.pallas_call(
        _route_body,
        out_shape=[jax.ShapeDtypeStruct((SEQ, 1), jnp.int32),
                   jax.ShapeDtypeStruct((SEQ, 1), jnp.int32),
                   jax.ShapeDtypeStruct((SEQ, WLANE), jnp.float32),
                   jax.ShapeDtypeStruct((SEQ, WLANE), jnp.float32),
                   jax.ShapeDtypeStruct((NBLK, 1), jnp.int32),
                   jax.ShapeDtypeStruct((1, 1), jnp.int32)],
    )(xs0, Wr, _tri_const())
    pos1 = pos1_2.reshape(SEQ)
    pos2 = pos2_2.reshape(SEQ)

    xs = _make_dispatch()(xs0, pos1, pos2)

    y = pl.pallas_call(
        _gemm_body,
        grid_spec=_GEMM_GRID,
        out_shape=jax.ShapeDtypeStruct((P, EMBED), jnp.float32),
    )(bexp_2, nact_2, xs, W1, b1.reshape(E, 1, FFN), W2, b2.reshape(E, 1, EMBED))

    out = _make_combine()(y, pos1, pos2, w1b, w2b)
    return out.reshape(x.shape)
